# Initial kernel scaffold; baseline (speedup 1.0000x reference)
import jax
import jax.numpy as jnp
from jax.experimental import pallas as pl


def kernel(x, edge_attr, edge_index, subgraphs_nodes_mapper, combined_subgraphs, subgraphs_edges_mapper, subgraphs_batch, mask, params):
    raise NotImplementedError("write your pallas kernel here")


# speedup vs baseline: 1.0232x; 1.0232x over previous
import functools

import jax
import jax.numpy as jnp
from jax import lax
from jax.experimental import pallas as pl
from jax.experimental.pallas import tpu as pltpu

F32 = jnp.float32


def _matmul_bias(x, w, b, block_rows, relu=False):
    n, k = x.shape
    m = w.shape[1]

    def body(x_ref, w_ref, b_ref, o_ref):
        acc = jnp.dot(x_ref[...], w_ref[...], preferred_element_type=F32)
        acc = acc + b_ref[...]
        if relu:
            acc = jnp.maximum(acc, 0.0)
        o_ref[...] = acc

    return pl.pallas_call(
        body,
        grid=(n // block_rows,),
        in_specs=[
            pl.BlockSpec((block_rows, k), lambda i: (i, 0)),
            pl.BlockSpec((k, m), lambda i: (0, 0)),
            pl.BlockSpec((1, m), lambda i: (0, 0)),
        ],
        out_specs=pl.BlockSpec((block_rows, m), lambda i: (i, 0)),
        out_shape=jax.ShapeDtypeStruct((n, m), F32),
    )(x, w, b.reshape(1, m))


def _gnn_mlp_stats(h, agg, w1, b1, w2, b2, eps):
    n, d = h.shape
    br = 2000

    def body(eps_ref, h_ref, a_ref, w1_ref, b1_ref, w2_ref, b2_ref,
             y_ref, st_ref):
        z = (1.0 + eps_ref[0]) * h_ref[...] + a_ref[...]
        t = jnp.dot(z, w1_ref[...], preferred_element_type=F32) + b1_ref[...]
        t = jnp.maximum(t, 0.0)
        y = jnp.dot(t, w2_ref[...], preferred_element_type=F32) + b2_ref[...]
        y_ref[...] = y

        @pl.when(pl.program_id(0) == 0)
        def _():
            st_ref[...] = jnp.zeros_like(st_ref)

        st_ref[0:1, :] += jnp.sum(y, axis=0, keepdims=True)
        st_ref[1:2, :] += jnp.sum(y * y, axis=0, keepdims=True)

    y, st = pl.pallas_call(
        body,
        grid=(n // br,),
        in_specs=[
            pl.BlockSpec(memory_space=pltpu.SMEM),
            pl.BlockSpec((br, d), lambda i: (i, 0)),
            pl.BlockSpec((br, d), lambda i: (i, 0)),
            pl.BlockSpec((d, d), lambda i: (0, 0)),
            pl.BlockSpec((1, d), lambda i: (0, 0)),
            pl.BlockSpec((d, d), lambda i: (0, 0)),
            pl.BlockSpec((1, d), lambda i: (0, 0)),
        ],
        out_specs=[
            pl.BlockSpec((br, d), lambda i: (i, 0)),
            pl.BlockSpec((8, d), lambda i: (0, 0)),
        ],
        out_shape=[
            jax.ShapeDtypeStruct((n, d), F32),
            jax.ShapeDtypeStruct((8, d), F32),
        ],
    )(eps.reshape(1), h, agg, w1, b1.reshape(1, d), w2, b2.reshape(1, d))
    return y, st


def _residual_bn_relu(h, y, scale, shift):
    n, d = h.shape
    br = 2000

    def body(h_ref, y_ref, sc_ref, sh_ref, o_ref):
        o_ref[...] = h_ref[...] + jnp.maximum(
            y_ref[...] * sc_ref[...] + sh_ref[...], 0.0)

    return pl.pallas_call(
        body,
        grid=(n // br,),
        in_specs=[
            pl.BlockSpec((br, d), lambda i: (i, 0)),
            pl.BlockSpec((br, d), lambda i: (i, 0)),
            pl.BlockSpec((1, d), lambda i: (0, 0)),
            pl.BlockSpec((1, d), lambda i: (0, 0)),
        ],
        out_specs=pl.BlockSpec((br, d), lambda i: (i, 0)),
        out_shape=jax.ShapeDtypeStruct((n, d), F32),
    )(h, y, scale.reshape(1, d), shift.reshape(1, d))


def _segsum_onehot(v, ids2d, nseg):
    n, d = v.shape
    br = 2000

    def body(ids_ref, v_ref, o_ref, c_ref):
        ids = ids_ref[...]
        seg = lax.broadcasted_iota(jnp.int32, (br, nseg), 1)
        oh = (ids == seg).astype(F32)
        part = lax.dot_general(oh, v_ref[...], (((0,), (0,)), ((), ())),
                               preferred_element_type=F32)

        @pl.when(pl.program_id(0) == 0)
        def _():
            o_ref[...] = jnp.zeros_like(o_ref)
            c_ref[...] = jnp.zeros_like(c_ref)

        o_ref[...] += part
        c_ref[0:1, :] += jnp.sum(oh, axis=0, keepdims=True)

    return pl.pallas_call(
        body,
        grid=(n // br,),
        in_specs=[
            pl.BlockSpec((br, 1), lambda i: (i, 0)),
            pl.BlockSpec((br, d), lambda i: (i, 0)),
        ],
        out_specs=[
            pl.BlockSpec((nseg, d), lambda i: (0, 0)),
            pl.BlockSpec((8, nseg), lambda i: (0, 0)),
        ],
        out_shape=[
            jax.ShapeDtypeStruct((nseg, d), F32),
            jax.ShapeDtypeStruct((8, nseg), F32),
        ],
    )(ids2d, v)


def _usub_relu(sums, invc, u_w, u_b):
    nseg, d = sums.shape

    def body(s_ref, ic_ref, w_ref, b_ref, o_ref):
        sub = s_ref[...] * ic_ref[...]
        o_ref[...] = jnp.maximum(
            jnp.dot(sub, w_ref[...], preferred_element_type=F32) + b_ref[...],
            0.0)

    return pl.pallas_call(
        body,
        out_shape=jax.ShapeDtypeStruct((nseg, d), F32),
    )(sums, invc, u_w, u_b.reshape(1, d))


def _ln_in(h, g, b):
    m = jnp.mean(h, axis=-1, keepdims=True)
    v = jnp.mean((h - m) ** 2, axis=-1, keepdims=True)
    return g * (h - m) * lax.rsqrt(v + 1e-5) + b


def _mixer_head(sums, invc, mfc, bsz, psz, p):
    nseg, d = sums.shape
    nlm = p['ln1_g'].shape[0]

    def body(s_ref, ic_ref, mf_ref, ln1g_ref, ln1b_ref, tw1_ref, tb1_ref,
             tw2_ref, tb2_ref, ln2g_ref, ln2b_ref, cw1_ref, cb1_ref,
             cw2_ref, cb2_ref, lnfg_ref, lnfb_ref, dw1_ref, db1_ref,
             dw2_ref, db2_ref, o_ref):
        mx = s_ref[...] * ic_ref[...]
        mfc_ = mf_ref[...]
        for i in range(nlm):
            t = _ln_in(mx, ln1g_ref[i], ln1b_ref[i]) * mfc_
            mixed = []
            for b in range(bsz):
                tb = t[b * psz:(b + 1) * psz, :]
                g1 = lax.dot_general(tb, tw1_ref[i],
                                     (((0,), (0,)), ((), ())),
                                     preferred_element_type=F32)
                g1 = jax.nn.gelu(g1 + tb1_ref[i])
                g2 = lax.dot_general(tw2_ref[i], g1,
                                     (((0,), (1,)), ((), ())),
                                     preferred_element_type=F32)
                mixed.append(g2 + tb2_ref[i][:, None])
            mx = mx + jnp.concatenate(mixed, axis=0)
            c = _ln_in(mx, ln2g_ref[i], ln2b_ref[i])
            ch = jax.nn.gelu(
                jnp.dot(c, cw1_ref[i], preferred_element_type=F32)
                + cb1_ref[i])
            mx = mx + (jnp.dot(ch, cw2_ref[i], preferred_element_type=F32)
                       + cb2_ref[i])
        mx = _ln_in(mx, lnfg_ref[...], lnfb_ref[...])
        pooled = []
        for b in range(bsz):
            mb = mfc_[b * psz:(b + 1) * psz, :]
            pr = jnp.sum(mx[b * psz:(b + 1) * psz, :] * mb, axis=0,
                         keepdims=True) / jnp.sum(mb)
            pooled.append(pr)
        pooled = jnp.concatenate(pooled, axis=0)
        hid = jnp.maximum(
            jnp.dot(pooled, dw1_ref[...], preferred_element_type=F32)
            + db1_ref[...], 0.0)
        o_ref[...] = (jnp.dot(hid, dw2_ref[...], preferred_element_type=F32)
                      + db2_ref[...])

    nout = p['dec_W2'].shape[1]
    return pl.pallas_call(
        body,
        out_shape=jax.ShapeDtypeStruct((bsz, nout), F32),
    )(sums, invc, mfc, p['ln1_g'], p['ln1_b'], p['tok_W1'], p['tok_b1'],
      p['tok_W2'], p['tok_b2'], p['ln2_g'], p['ln2_b'], p['ch_W1'],
      p['ch_b1'], p['ch_W2'], p['ch_b2'], p['lnf_g'], p['lnf_b'],
      p['dec_W1'], p['dec_b1'].reshape(1, -1), p['dec_W2'],
      p['dec_b2'].reshape(1, -1))


def kernel(x, edge_attr, edge_index, subgraphs_nodes_mapper,
           combined_subgraphs, subgraphs_edges_mapper, subgraphs_batch,
           mask, params):
    p = params
    nm = subgraphs_nodes_mapper
    em = subgraphs_edges_mapper
    batch = subgraphs_batch
    n_nodes, d = x.shape
    n_sub = nm.shape[0]
    e_cnt = edge_attr.shape[0]
    bsz, psz = mask.shape
    nseg = bsz * psz
    nlg = p['gnn_W1'].shape[0]

    h0 = _matmul_bias(x, p['W_in'], p['b_in'], 2000)
    ea = jnp.take(edge_attr, em, axis=0)
    e = _matmul_bias(ea, p['W_edge'], p['b_edge'], 4000)
    h = jnp.take(h0, nm, axis=0)

    src = combined_subgraphs[0]
    dst = combined_subgraphs[1]
    batch2d = batch.astype(jnp.int32).reshape(n_sub, 1)
    nm32 = nm.astype(jnp.int32)

    for i in range(nlg):
        if i > 0:
            sums, cst = _segsum_onehot(h, batch2d, nseg)
            invc = (1.0 / jnp.maximum(cst[0], 1.0)).reshape(nseg, 1)
            usub = _usub_relu(sums, invc, p['U_W'][i - 1], p['U_b'][i - 1])
            h = h + jnp.take(usub, batch, axis=0)
            s = jax.ops.segment_sum(h, nm32, num_segments=n_nodes)
            c = jax.ops.segment_sum(jnp.ones((n_sub,), F32), nm32,
                                    num_segments=n_nodes)
            h = jnp.take(s / jnp.maximum(c, 1.0)[:, None], nm, axis=0)
        m = jax.nn.relu(jnp.take(h, src, axis=0) + e)
        agg = jax.ops.segment_sum(m, dst, num_segments=n_sub)
        y, st = _gnn_mlp_stats(h, agg, p['gnn_W1'][i], p['gnn_b1'][i],
                               p['gnn_W2'][i], p['gnn_b2'][i],
                               p['gnn_eps'][i])
        mean = st[0] / n_sub
        var = st[1] / n_sub - mean * mean
        scale = p['bn_g'][i] * lax.rsqrt(var + 1e-5)
        shift = p['bn_b'][i] - mean * scale
        h = _residual_bn_relu(h, y, scale, shift)

    sums, cst = _segsum_onehot(h, batch2d, nseg)
    invc = (1.0 / jnp.maximum(cst[0], 1.0)).reshape(nseg, 1)
    mfc = mask.astype(F32).reshape(nseg, 1)
    return _mixer_head(sums, invc, mfc, bsz, psz, p)


# speedup vs baseline: 1.1744x; 1.1477x over previous
import functools

import jax
import jax.numpy as jnp
from jax import lax
from jax.experimental import pallas as pl
from jax.experimental.pallas import tpu as pltpu
from jax.experimental.pallas import tpu_sc as plsc

F32 = jnp.float32

_SC = plsc.get_sparse_core_info()
_NC, _NS = _SC.num_cores, _SC.num_subcores
_NW = _NC * _NS
_BLK = 128


def _sc_gather(table, idx):
    mp = idx.shape[0]
    n, d = table.shape
    per_w = mp // _NW
    nwin = per_w // _BLK
    mesh = plsc.VectorSubcoreMesh(core_axis_name="c", subcore_axis_name="s")

    @functools.partial(
        pl.kernel, mesh=mesh,
        out_type=jax.ShapeDtypeStruct((mp, d), F32),
        scratch_types=[
            pltpu.VMEM((_BLK,), jnp.int32),
            pltpu.VMEM((_BLK, d), F32),
            pltpu.SemaphoreType.DMA,
        ],
    )
    def k(table_hbm, idx_hbm, out_hbm, idx_v, rows_v, sem):
        wid = lax.axis_index("s") * _NC + lax.axis_index("c")
        base = wid * per_w

        def step(w, carry):
            off = base + w * _BLK
            pltpu.sync_copy(idx_hbm.at[pl.ds(off, _BLK)], idx_v)
            pltpu.async_copy(table_hbm.at[idx_v], rows_v, sem).wait()
            pltpu.sync_copy(rows_v, out_hbm.at[pl.ds(off, _BLK)])
            return carry

        lax.fori_loop(0, nwin, step, 0)

    return k(table, idx)


def _sc_scatter_partials(values, idx, ones_blk, np_rows, use_ones):
    mp = idx.shape[0]
    d = ones_blk.shape[1]
    per_w = mp // _NW
    nwin = per_w // _BLK
    tile_rows = np_rows // _NS
    mesh = plsc.VectorSubcoreMesh(core_axis_name="c", subcore_axis_name="s")

    @functools.partial(
        pl.kernel, mesh=mesh,
        out_type=jax.ShapeDtypeStruct((_NC, np_rows, d), F32),
        scratch_types=[
            pltpu.VMEM((1, _BLK), jnp.int32),
            pltpu.VMEM((_BLK, d), F32),
            pltpu.VMEM_SHARED((np_rows, d), F32),
            pltpu.SemaphoreType.DMA,
        ],
    )
    def k(val_hbm, idx_hbm, ones_hbm, zero_hbm, out_hbm,
          idx_v, val_v, acc_s, sem):
        cid = lax.axis_index("c")
        sid = lax.axis_index("s")
        wid = sid * _NC + cid
        pltpu.sync_copy(zero_hbm.at[pl.ds(sid * tile_rows, tile_rows)],
                        acc_s.at[pl.ds(sid * tile_rows, tile_rows)])
        if use_ones:
            pltpu.sync_copy(ones_hbm, val_v)
        plsc.subcore_barrier()

        def step(w, carry):
            off = wid * per_w + w * _BLK
            pltpu.sync_copy(idx_hbm.at[pl.ds(off, _BLK)], idx_v.at[0])
            if not use_ones:
                pltpu.sync_copy(val_hbm.at[pl.ds(off, _BLK)], val_v)
            pltpu.sync_copy(val_v, acc_s.at[idx_v.at[0]], add=True)
            return carry

        lax.fori_loop(0, nwin, step, 0)
        plsc.subcore_barrier()
        pltpu.sync_copy(acc_s.at[pl.ds(sid * tile_rows, tile_rows)],
                        out_hbm.at[cid, pl.ds(sid * tile_rows, tile_rows)])

    zeros = jnp.zeros((np_rows, d), F32)
    return k(values, idx, ones_blk, zeros)


def _matmul_bias(x, w, b, block_rows, relu=False):
    n, k = x.shape
    m = w.shape[1]

    def body(x_ref, w_ref, b_ref, o_ref):
        acc = jnp.dot(x_ref[...], w_ref[...], preferred_element_type=F32)
        acc = acc + b_ref[...]
        if relu:
            acc = jnp.maximum(acc, 0.0)
        o_ref[...] = acc

    return pl.pallas_call(
        body,
        grid=(n // block_rows,),
        in_specs=[
            pl.BlockSpec((block_rows, k), lambda i: (i, 0)),
            pl.BlockSpec((k, m), lambda i: (0, 0)),
            pl.BlockSpec((1, m), lambda i: (0, 0)),
        ],
        out_specs=pl.BlockSpec((block_rows, m), lambda i: (i, 0)),
        out_shape=jax.ShapeDtypeStruct((n, m), F32),
    )(x, w, b.reshape(1, m))


def _gnn_mlp_stats(h, agg, w1, b1, w2, b2, eps, n_valid):
    n, d = h.shape
    br = 2048

    def body(eps_ref, h_ref, a_ref, w1_ref, b1_ref, w2_ref, b2_ref,
             y_ref, st_ref):
        z = (1.0 + eps_ref[0]) * h_ref[...] + a_ref[...]
        t = jnp.dot(z, w1_ref[...], preferred_element_type=F32) + b1_ref[...]
        t = jnp.maximum(t, 0.0)
        y = jnp.dot(t, w2_ref[...], preferred_element_type=F32) + b2_ref[...]
        y_ref[...] = y

        @pl.when(pl.program_id(0) == 0)
        def _():
            st_ref[...] = jnp.zeros_like(st_ref)

        rid = pl.program_id(0) * br + lax.broadcasted_iota(
            jnp.int32, (br, 1), 0)
        ym = jnp.where(rid < n_valid, y, 0.0)
        st_ref[0:1, :] += jnp.sum(ym, axis=0, keepdims=True)
        st_ref[1:2, :] += jnp.sum(ym * y, axis=0, keepdims=True)

    y, st = pl.pallas_call(
        body,
        grid=(n // br,),
        in_specs=[
            pl.BlockSpec(memory_space=pltpu.SMEM),
            pl.BlockSpec((br, d), lambda i: (i, 0)),
            pl.BlockSpec((br, d), lambda i: (i, 0)),
            pl.BlockSpec((d, d), lambda i: (0, 0)),
            pl.BlockSpec((1, d), lambda i: (0, 0)),
            pl.BlockSpec((d, d), lambda i: (0, 0)),
            pl.BlockSpec((1, d), lambda i: (0, 0)),
        ],
        out_specs=[
            pl.BlockSpec((br, d), lambda i: (i, 0)),
            pl.BlockSpec((8, d), lambda i: (0, 0)),
        ],
        out_shape=[
            jax.ShapeDtypeStruct((n, d), F32),
            jax.ShapeDtypeStruct((8, d), F32),
        ],
    )(eps.reshape(1), h, agg, w1, b1.reshape(1, d), w2, b2.reshape(1, d))
    return y, st


def _residual_bn_relu(h, y, scale, shift):
    n, d = h.shape
    br = 2048

    def body(h_ref, y_ref, sc_ref, sh_ref, o_ref):
        o_ref[...] = h_ref[...] + jnp.maximum(
            y_ref[...] * sc_ref[...] + sh_ref[...], 0.0)

    return pl.pallas_call(
        body,
        grid=(n // br,),
        in_specs=[
            pl.BlockSpec((br, d), lambda i: (i, 0)),
            pl.BlockSpec((br, d), lambda i: (i, 0)),
            pl.BlockSpec((1, d), lambda i: (0, 0)),
            pl.BlockSpec((1, d), lambda i: (0, 0)),
        ],
        out_specs=pl.BlockSpec((br, d), lambda i: (i, 0)),
        out_shape=jax.ShapeDtypeStruct((n, d), F32),
    )(h, y, scale.reshape(1, d), shift.reshape(1, d))


def _segsum_onehot(v, ids2d, nseg):
    n, d = v.shape
    br = 2048

    def body(ids_ref, v_ref, o_ref, c_ref):
        ids = ids_ref[...]
        seg = lax.broadcasted_iota(jnp.int32, (br, nseg), 1)
        oh = (ids == seg).astype(F32)
        part = lax.dot_general(oh, v_ref[...], (((0,), (0,)), ((), ())),
                               preferred_element_type=F32)

        @pl.when(pl.program_id(0) == 0)
        def _():
            o_ref[...] = jnp.zeros_like(o_ref)
            c_ref[...] = jnp.zeros_like(c_ref)

        o_ref[...] += part
        c_ref[0:1, :] += jnp.sum(oh, axis=0, keepdims=True)

    return pl.pallas_call(
        body,
        grid=(n // br,),
        in_specs=[
            pl.BlockSpec((br, 1), lambda i: (i, 0)),
            pl.BlockSpec((br, d), lambda i: (i, 0)),
        ],
        out_specs=[
            pl.BlockSpec((nseg, d), lambda i: (0, 0)),
            pl.BlockSpec((8, nseg), lambda i: (0, 0)),
        ],
        out_shape=[
            jax.ShapeDtypeStruct((nseg, d), F32),
            jax.ShapeDtypeStruct((8, nseg), F32),
        ],
    )(ids2d, v)


def _expand_add(h, ids2d, table):
    n, d = h.shape
    nseg = table.shape[0]
    br = 2048

    def body(ids_ref, h_ref, t_ref, o_ref):
        seg = lax.broadcasted_iota(jnp.int32, (br, nseg), 1)
        oh = (ids_ref[...] == seg).astype(F32)
        o_ref[...] = h_ref[...] + lax.dot_general(
            oh, t_ref[...], (((1,), (0,)), ((), ())),
            preferred_element_type=F32)

    return pl.pallas_call(
        body,
        grid=(n // br,),
        in_specs=[
            pl.BlockSpec((br, 1), lambda i: (i, 0)),
            pl.BlockSpec((br, d), lambda i: (i, 0)),
            pl.BlockSpec((nseg, d), lambda i: (0, 0)),
        ],
        out_specs=pl.BlockSpec((br, d), lambda i: (i, 0)),
        out_shape=jax.ShapeDtypeStruct((n, d), F32),
    )(ids2d, h, table)


def _combine_mean_table(p0, p1, c0, c1):
    n, d = p0.shape
    br = 2048

    def body(p0_ref, p1_ref, c0_ref, c1_ref, o_ref):
        cnt = c0_ref[...][:, 0:1] + c1_ref[...][:, 0:1]
        o_ref[...] = (p0_ref[...] + p1_ref[...]) / jnp.maximum(cnt, 1.0)

    return pl.pallas_call(
        body,
        grid=(n // br,),
        in_specs=[pl.BlockSpec((br, d), lambda i: (i, 0))] * 4,
        out_specs=pl.BlockSpec((br, d), lambda i: (i, 0)),
        out_shape=jax.ShapeDtypeStruct((n, d), F32),
    )(p0, p1, c0, c1)


def _usub_relu(sums, invc, u_w, u_b):
    nseg, d = sums.shape

    def body(s_ref, ic_ref, w_ref, b_ref, o_ref):
        sub = s_ref[...] * ic_ref[...]
        o_ref[...] = jnp.maximum(
            jnp.dot(sub, w_ref[...], preferred_element_type=F32) + b_ref[...],
            0.0)

    return pl.pallas_call(
        body,
        out_shape=jax.ShapeDtypeStruct((nseg, d), F32),
    )(sums, invc, u_w, u_b.reshape(1, d))


def _ln_in(h, g, b):
    m = jnp.mean(h, axis=-1, keepdims=True)
    v = jnp.mean((h - m) ** 2, axis=-1, keepdims=True)
    return g * (h - m) * lax.rsqrt(v + 1e-5) + b


def _mixer_head(sums, invc, mfc, bsz, psz, p):
    nseg, d = sums.shape
    nlm = p['ln1_g'].shape[0]

    def body(s_ref, ic_ref, mf_ref, ln1g_ref, ln1b_ref, tw1_ref, tb1_ref,
             tw2_ref, tb2_ref, ln2g_ref, ln2b_ref, cw1_ref, cb1_ref,
             cw2_ref, cb2_ref, lnfg_ref, lnfb_ref, dw1_ref, db1_ref,
             dw2_ref, db2_ref, o_ref):
        mx = s_ref[...] * ic_ref[...]
        mfc_ = mf_ref[...]
        for i in range(nlm):
            t = _ln_in(mx, ln1g_ref[i], ln1b_ref[i]) * mfc_
            mixed = []
            for b in range(bsz):
                tb = t[b * psz:(b + 1) * psz, :]
                g1 = lax.dot_general(tb, tw1_ref[i],
                                     (((0,), (0,)), ((), ())),
                                     preferred_element_type=F32)
                g1 = jax.nn.gelu(g1 + tb1_ref[i])
                g2 = lax.dot_general(tw2_ref[i], g1,
                                     (((0,), (1,)), ((), ())),
                                     preferred_element_type=F32)
                mixed.append(g2 + tb2_ref[i][:, None])
            mx = mx + jnp.concatenate(mixed, axis=0)
            c = _ln_in(mx, ln2g_ref[i], ln2b_ref[i])
            ch = jax.nn.gelu(
                jnp.dot(c, cw1_ref[i], preferred_element_type=F32)
                + cb1_ref[i])
            mx = mx + (jnp.dot(ch, cw2_ref[i], preferred_element_type=F32)
                       + cb2_ref[i])
        mx = _ln_in(mx, lnfg_ref[...], lnfb_ref[...])
        pooled = []
        for b in range(bsz):
            mb = mfc_[b * psz:(b + 1) * psz, :]
            pr = jnp.sum(mx[b * psz:(b + 1) * psz, :] * mb, axis=0,
                         keepdims=True) / jnp.sum(mb)
            pooled.append(pr)
        pooled = jnp.concatenate(pooled, axis=0)
        hid = jnp.maximum(
            jnp.dot(pooled, dw1_ref[...], preferred_element_type=F32)
            + db1_ref[...], 0.0)
        o_ref[...] = (jnp.dot(hid, dw2_ref[...], preferred_element_type=F32)
                      + db2_ref[...])

    nout = p['dec_W2'].shape[1]
    return pl.pallas_call(
        body,
        out_shape=jax.ShapeDtypeStruct((bsz, nout), F32),
    )(sums, invc, mfc, p['ln1_g'], p['ln1_b'], p['tok_W1'], p['tok_b1'],
      p['tok_W2'], p['tok_b2'], p['ln2_g'], p['ln2_b'], p['ch_W1'],
      p['ch_b1'], p['ch_W2'], p['ch_b2'], p['lnf_g'], p['lnf_b'],
      p['dec_W1'], p['dec_b1'].reshape(1, -1), p['dec_W2'],
      p['dec_b2'].reshape(1, -1))


def kernel(x, edge_attr, edge_index, subgraphs_nodes_mapper,
           combined_subgraphs, subgraphs_edges_mapper, subgraphs_batch,
           mask, params):
    p = params
    nm = subgraphs_nodes_mapper
    em = subgraphs_edges_mapper
    batch = subgraphs_batch
    n_nodes, d = x.shape
    n_sub = nm.shape[0]
    e_cnt = edge_attr.shape[0]
    bsz, psz = mask.shape
    nseg = bsz * psz
    nlg = p['gnn_W1'].shape[0]

    align = _NW * _BLK * 25
    nsubp = ((n_sub + align - 1) // align) * align
    npad = nsubp - n_sub
    np_rows = ((n_nodes + 2047) // 2048 + 1) * 2048

    nm32 = nm.astype(jnp.int32)
    nm_g = jnp.concatenate([nm32, jnp.zeros((npad,), jnp.int32)])
    dummies = n_nodes + (jnp.arange(npad, dtype=jnp.int32)
                         % (np_rows - n_nodes))
    nm_s = jnp.concatenate([nm32, dummies])
    batch2d = jnp.concatenate(
        [batch.astype(jnp.int32),
         jnp.full((npad,), nseg, jnp.int32)]).reshape(nsubp, 1)
    ones_blk = jnp.ones((_BLK, d), F32)

    h0 = _matmul_bias(x, p['W_in'], p['b_in'], 2000)
    ea = jnp.take(edge_attr, em, axis=0)
    e = _matmul_bias(ea, p['W_edge'], p['b_edge'], 4000)
    h = _sc_gather(h0, nm_g)

    src = combined_subgraphs[0]
    dst = combined_subgraphs[1]

    cparts = None
    for i in range(nlg):
        if i > 0:
            sums, cst = _segsum_onehot(h, batch2d, nseg)
            invc = (1.0 / jnp.maximum(cst[0], 1.0)).reshape(nseg, 1)
            usub = _usub_relu(sums, invc, p['U_W'][i - 1], p['U_b'][i - 1])
            hp = _expand_add(h, batch2d, usub)
            parts = _sc_scatter_partials(hp, nm_s, ones_blk, np_rows, False)
            if cparts is None:
                cparts = _sc_scatter_partials(hp, nm_s, ones_blk, np_rows,
                                              True)
            table = _combine_mean_table(parts[0], parts[1],
                                        cparts[0], cparts[1])
            h = _sc_gather(table, nm_g)
        m = jax.nn.relu(jnp.take(h, src, axis=0) + e)
        agg = jax.ops.segment_sum(m, dst, num_segments=nsubp)
        y, st = _gnn_mlp_stats(h, agg, p['gnn_W1'][i], p['gnn_b1'][i],
                               p['gnn_W2'][i], p['gnn_b2'][i],
                               p['gnn_eps'][i], n_sub)
        mean = st[0] / n_sub
        var = st[1] / n_sub - mean * mean
        scale = p['bn_g'][i] * lax.rsqrt(var + 1e-5)
        shift = p['bn_b'][i] - mean * scale
        h = _residual_bn_relu(h, y, scale, shift)

    sums, cst = _segsum_onehot(h, batch2d, nseg)
    invc = (1.0 / jnp.maximum(cst[0], 1.0)).reshape(nseg, 1)
    mfc = mask.astype(F32).reshape(nseg, 1)
    return _mixer_head(sums, invc, mfc, bsz, psz, p)


# speedup vs baseline: 1.2980x; 1.1053x over previous
import functools

import jax
import jax.numpy as jnp
from jax import lax
from jax.experimental import pallas as pl
from jax.experimental.pallas import tpu as pltpu
from jax.experimental.pallas import tpu_sc as plsc

F32 = jnp.float32

_SC = plsc.get_sparse_core_info()
_NC, _NS = _SC.num_cores, _SC.num_subcores
_NW = _NC * _NS
_BLK = 128
_CH = 6400


def _sc_gather(table, idx):
    mp = idx.shape[0]
    n, d = table.shape
    per_w = mp // _NW
    nwin = per_w // _BLK
    mesh = plsc.VectorSubcoreMesh(core_axis_name="c", subcore_axis_name="s")

    @functools.partial(
        pl.kernel, mesh=mesh,
        out_type=jax.ShapeDtypeStruct((mp, d), F32),
        scratch_types=[
            pltpu.VMEM((_BLK,), jnp.int32),
            pltpu.VMEM((_BLK, d), F32),
            pltpu.SemaphoreType.DMA,
        ],
    )
    def k(table_hbm, idx_hbm, out_hbm, idx_v, rows_v, sem):
        wid = lax.axis_index("s") * _NC + lax.axis_index("c")
        base = wid * per_w

        def step(w, carry):
            off = base + w * _BLK
            pltpu.sync_copy(idx_hbm.at[pl.ds(off, _BLK)], idx_v)
            pltpu.async_copy(table_hbm.at[idx_v], rows_v, sem).wait()
            pltpu.sync_copy(rows_v, out_hbm.at[pl.ds(off, _BLK)])
            return carry

        lax.fori_loop(0, nwin, step, 0)

    return k(table, idx)


def _sc_scatter_partials(values, idx, ones_blk, np_rows, use_ones):
    mp = idx.shape[0]
    d = ones_blk.shape[1]
    per_w = mp // _NW
    nwin = per_w // _BLK
    tile_rows = np_rows // _NS
    mesh = plsc.VectorSubcoreMesh(core_axis_name="c", subcore_axis_name="s")

    @functools.partial(
        pl.kernel, mesh=mesh,
        out_type=jax.ShapeDtypeStruct((_NC, np_rows, d), F32),
        scratch_types=[
            pltpu.VMEM((1, _BLK), jnp.int32),
            pltpu.VMEM((_BLK, d), F32),
            pltpu.VMEM_SHARED((np_rows, d), F32),
            pltpu.SemaphoreType.DMA,
        ],
    )
    def k(val_hbm, idx_hbm, ones_hbm, zero_hbm, out_hbm,
          idx_v, val_v, acc_s, sem):
        cid = lax.axis_index("c")
        sid = lax.axis_index("s")
        wid = sid * _NC + cid
        pltpu.sync_copy(zero_hbm.at[pl.ds(sid * tile_rows, tile_rows)],
                        acc_s.at[pl.ds(sid * tile_rows, tile_rows)])
        if use_ones:
            pltpu.sync_copy(ones_hbm, val_v)
        plsc.subcore_barrier()

        def step(w, carry):
            off = wid * per_w + w * _BLK
            pltpu.sync_copy(idx_hbm.at[pl.ds(off, _BLK)], idx_v.at[0])
            if not use_ones:
                pltpu.sync_copy(val_hbm.at[pl.ds(off, _BLK)], val_v)
            pltpu.sync_copy(val_v, acc_s.at[idx_v.at[0]], add=True)
            return carry

        lax.fori_loop(0, nwin, step, 0)
        plsc.subcore_barrier()
        pltpu.sync_copy(acc_s.at[pl.ds(sid * tile_rows, tile_rows)],
                        out_hbm.at[cid, pl.ds(sid * tile_rows, tile_rows)])

    zeros = jnp.zeros((np_rows, d), F32)
    return k(values, idx, ones_blk, zeros)


def _sc_message_agg(h, e_enc, src_s, dmod_s, em_s, blo, bhi, nsubp):
    d = h.shape[1]
    ep = src_s.shape[0]
    nchunk = nsubp // _CH
    ncpc = nchunk // _NC
    acc_rows = _CH + 256
    zslab = acc_rows // _NS
    blk = 256
    mesh = plsc.VectorSubcoreMesh(core_axis_name="c", subcore_axis_name="s")

    @functools.partial(
        pl.kernel, mesh=mesh,
        out_type=jax.ShapeDtypeStruct((nsubp, d), F32),
        scratch_types=[
            pltpu.VMEM((32,), jnp.int32),
            pltpu.VMEM((32,), jnp.int32),
            pltpu.VMEM((blk,), jnp.int32),
            pltpu.VMEM((blk,), jnp.int32),
            pltpu.VMEM((2, 128), jnp.int32),
            pltpu.VMEM((blk, 128), F32),
            pltpu.VMEM((blk, 128), F32),
            pltpu.VMEM_SHARED((acc_rows, d), F32),
            pltpu.SemaphoreType.DMA,
        ],
    )
    def k(h_hbm, e_hbm, src_hbm, dmod_hbm, em_hbm, blo_hbm, bhi_hbm,
          zero_hbm, out_hbm, blov, bhiv, sidx, eidx, didx,
          hbuf, ebuf, acc_s, sem):
        cid = lax.axis_index("c")
        sid = lax.axis_index("s")
        pltpu.sync_copy(blo_hbm, blov)
        pltpu.sync_copy(bhi_hbm, bhiv)

        def chunk_body(q, carry0):
            cc = cid * ncpc + q
            lo = cc * _CH
            pltpu.sync_copy(zero_hbm.at[pl.ds(sid * zslab, zslab)],
                            acc_s.at[pl.ds(sid * zslab, zslab)])
            plsc.subcore_barrier()
            k0 = blov[pl.ds(cc, 16)][0] + sid
            k1 = bhiv[pl.ds(cc, 16)][0]
            ntrip = (k1 - k0 + _NS - 1) // _NS

            def block_body(j, carry1):
                base = (k0 + j * _NS) * blk
                pltpu.sync_copy(src_hbm.at[pl.ds(base, blk)], sidx)
                pltpu.sync_copy(em_hbm.at[pl.ds(base, blk)], eidx)
                for half in range(2):
                    pltpu.sync_copy(
                        dmod_hbm.at[pl.ds(base + half * 128, 128)],
                        didx.at[half])
                cps = []
                for half in range(2):
                    cps.append(pltpu.async_copy(
                        h_hbm.at[sidx.at[pl.ds(half * 128, 128)]],
                        hbuf.at[pl.ds(half * 128, 128)], sem))
                    cps.append(pltpu.async_copy(
                        e_hbm.at[eidx.at[pl.ds(half * 128, 128)]],
                        ebuf.at[pl.ds(half * 128, 128)], sem))
                for cp in cps:
                    cp.wait()

                def rowcomp(r, c2):
                    for cc2 in range(8):
                        hv = hbuf[r, pl.ds(cc2 * 16, 16)]
                        ev = ebuf[r, pl.ds(cc2 * 16, 16)]
                        hbuf[r, pl.ds(cc2 * 16, 16)] = jnp.maximum(
                            hv + ev, 0.0)
                    return c2

                lax.fori_loop(0, blk, rowcomp, 0)
                for half in range(2):
                    pltpu.sync_copy(hbuf.at[pl.ds(half * 128, 128)],
                                    acc_s.at[didx.at[half]], add=True)
                return carry1

            lax.fori_loop(0, ntrip, block_body, 0)
            plsc.subcore_barrier()
            pltpu.sync_copy(
                acc_s.at[pl.ds(sid * (_CH // _NS), _CH // _NS)],
                out_hbm.at[pl.ds(lo + sid * (_CH // _NS), _CH // _NS)])
            plsc.subcore_barrier()
            return carry0

        lax.fori_loop(0, ncpc, chunk_body, 0)

    zeros = jnp.zeros((acc_rows, d), F32)
    return k(h, e_enc, src_s, dmod_s, em_s, blo, bhi, zeros)


def _matmul_bias(x, w, b, block_rows, relu=False):
    n, k = x.shape
    m = w.shape[1]

    def body(x_ref, w_ref, b_ref, o_ref):
        acc = jnp.dot(x_ref[...], w_ref[...], preferred_element_type=F32)
        acc = acc + b_ref[...]
        if relu:
            acc = jnp.maximum(acc, 0.0)
        o_ref[...] = acc

    return pl.pallas_call(
        body,
        grid=(n // block_rows,),
        in_specs=[
            pl.BlockSpec((block_rows, k), lambda i: (i, 0)),
            pl.BlockSpec((k, m), lambda i: (0, 0)),
            pl.BlockSpec((1, m), lambda i: (0, 0)),
        ],
        out_specs=pl.BlockSpec((block_rows, m), lambda i: (i, 0)),
        out_shape=jax.ShapeDtypeStruct((n, m), F32),
    )(x, w, b.reshape(1, m))


def _gnn_mlp_stats(h, agg, w1, b1, w2, b2, eps, n_valid):
    n, d = h.shape
    br = 2048

    def body(eps_ref, h_ref, a_ref, w1_ref, b1_ref, w2_ref, b2_ref,
             y_ref, st_ref):
        z = (1.0 + eps_ref[0]) * h_ref[...] + a_ref[...]
        t = jnp.dot(z, w1_ref[...], preferred_element_type=F32) + b1_ref[...]
        t = jnp.maximum(t, 0.0)
        y = jnp.dot(t, w2_ref[...], preferred_element_type=F32) + b2_ref[...]
        y_ref[...] = y

        @pl.when(pl.program_id(0) == 0)
        def _():
            st_ref[...] = jnp.zeros_like(st_ref)

        rid = pl.program_id(0) * br + lax.broadcasted_iota(
            jnp.int32, (br, 1), 0)
        ym = jnp.where(rid < n_valid, y, 0.0)
        st_ref[0:1, :] += jnp.sum(ym, axis=0, keepdims=True)
        st_ref[1:2, :] += jnp.sum(ym * y, axis=0, keepdims=True)

    y, st = pl.pallas_call(
        body,
        grid=(n // br,),
        in_specs=[
            pl.BlockSpec(memory_space=pltpu.SMEM),
            pl.BlockSpec((br, d), lambda i: (i, 0)),
            pl.BlockSpec((br, d), lambda i: (i, 0)),
            pl.BlockSpec((d, d), lambda i: (0, 0)),
            pl.BlockSpec((1, d), lambda i: (0, 0)),
            pl.BlockSpec((d, d), lambda i: (0, 0)),
            pl.BlockSpec((1, d), lambda i: (0, 0)),
        ],
        out_specs=[
            pl.BlockSpec((br, d), lambda i: (i, 0)),
            pl.BlockSpec((8, d), lambda i: (0, 0)),
        ],
        out_shape=[
            jax.ShapeDtypeStruct((n, d), F32),
            jax.ShapeDtypeStruct((8, d), F32),
        ],
    )(eps.reshape(1), h, agg, w1, b1.reshape(1, d), w2, b2.reshape(1, d))
    return y, st


def _residual_bn_relu(h, y, scale, shift):
    n, d = h.shape
    br = 2048

    def body(h_ref, y_ref, sc_ref, sh_ref, o_ref):
        o_ref[...] = h_ref[...] + jnp.maximum(
            y_ref[...] * sc_ref[...] + sh_ref[...], 0.0)

    return pl.pallas_call(
        body,
        grid=(n // br,),
        in_specs=[
            pl.BlockSpec((br, d), lambda i: (i, 0)),
            pl.BlockSpec((br, d), lambda i: (i, 0)),
            pl.BlockSpec((1, d), lambda i: (0, 0)),
            pl.BlockSpec((1, d), lambda i: (0, 0)),
        ],
        out_specs=pl.BlockSpec((br, d), lambda i: (i, 0)),
        out_shape=jax.ShapeDtypeStruct((n, d), F32),
    )(h, y, scale.reshape(1, d), shift.reshape(1, d))


def _segsum_onehot(v, ids2d, nseg):
    n, d = v.shape
    br = 2048

    def body(ids_ref, v_ref, o_ref, c_ref):
        ids = ids_ref[...]
        seg = lax.broadcasted_iota(jnp.int32, (br, nseg), 1)
        oh = (ids == seg).astype(F32)
        part = lax.dot_general(oh, v_ref[...], (((0,), (0,)), ((), ())),
                               preferred_element_type=F32)

        @pl.when(pl.program_id(0) == 0)
        def _():
            o_ref[...] = jnp.zeros_like(o_ref)
            c_ref[...] = jnp.zeros_like(c_ref)

        o_ref[...] += part
        c_ref[0:1, :] += jnp.sum(oh, axis=0, keepdims=True)

    return pl.pallas_call(
        body,
        grid=(n // br,),
        in_specs=[
            pl.BlockSpec((br, 1), lambda i: (i, 0)),
            pl.BlockSpec((br, d), lambda i: (i, 0)),
        ],
        out_specs=[
            pl.BlockSpec((nseg, d), lambda i: (0, 0)),
            pl.BlockSpec((8, nseg), lambda i: (0, 0)),
        ],
        out_shape=[
            jax.ShapeDtypeStruct((nseg, d), F32),
            jax.ShapeDtypeStruct((8, nseg), F32),
        ],
    )(ids2d, v)


def _expand_add(h, ids2d, table):
    n, d = h.shape
    nseg = table.shape[0]
    br = 2048

    def body(ids_ref, h_ref, t_ref, o_ref):
        seg = lax.broadcasted_iota(jnp.int32, (br, nseg), 1)
        oh = (ids_ref[...] == seg).astype(F32)
        o_ref[...] = h_ref[...] + lax.dot_general(
            oh, t_ref[...], (((1,), (0,)), ((), ())),
            preferred_element_type=F32)

    return pl.pallas_call(
        body,
        grid=(n // br,),
        in_specs=[
            pl.BlockSpec((br, 1), lambda i: (i, 0)),
            pl.BlockSpec((br, d), lambda i: (i, 0)),
            pl.BlockSpec((nseg, d), lambda i: (0, 0)),
        ],
        out_specs=pl.BlockSpec((br, d), lambda i: (i, 0)),
        out_shape=jax.ShapeDtypeStruct((n, d), F32),
    )(ids2d, h, table)


def _combine_mean_table(p0, p1, c0, c1):
    n, d = p0.shape
    br = 2048

    def body(p0_ref, p1_ref, c0_ref, c1_ref, o_ref):
        cnt = c0_ref[...][:, 0:1] + c1_ref[...][:, 0:1]
        o_ref[...] = (p0_ref[...] + p1_ref[...]) / jnp.maximum(cnt, 1.0)

    return pl.pallas_call(
        body,
        grid=(n // br,),
        in_specs=[pl.BlockSpec((br, d), lambda i: (i, 0))] * 4,
        out_specs=pl.BlockSpec((br, d), lambda i: (i, 0)),
        out_shape=jax.ShapeDtypeStruct((n, d), F32),
    )(p0, p1, c0, c1)


def _usub_relu(sums, invc, u_w, u_b):
    nseg, d = sums.shape

    def body(s_ref, ic_ref, w_ref, b_ref, o_ref):
        sub = s_ref[...] * ic_ref[...]
        o_ref[...] = jnp.maximum(
            jnp.dot(sub, w_ref[...], preferred_element_type=F32) + b_ref[...],
            0.0)

    return pl.pallas_call(
        body,
        out_shape=jax.ShapeDtypeStruct((nseg, d), F32),
    )(sums, invc, u_w, u_b.reshape(1, d))


def _ln_in(h, g, b):
    m = jnp.mean(h, axis=-1, keepdims=True)
    v = jnp.mean((h - m) ** 2, axis=-1, keepdims=True)
    return g * (h - m) * lax.rsqrt(v + 1e-5) + b


def _mixer_head(sums, invc, mfc, bsz, psz, p):
    nseg, d = sums.shape
    nlm = p['ln1_g'].shape[0]

    def body(s_ref, ic_ref, mf_ref, ln1g_ref, ln1b_ref, tw1_ref, tb1_ref,
             tw2_ref, tb2_ref, ln2g_ref, ln2b_ref, cw1_ref, cb1_ref,
             cw2_ref, cb2_ref, lnfg_ref, lnfb_ref, dw1_ref, db1_ref,
             dw2_ref, db2_ref, o_ref):
        mx = s_ref[...] * ic_ref[...]
        mfc_ = mf_ref[...]
        for i in range(nlm):
            t = _ln_in(mx, ln1g_ref[i], ln1b_ref[i]) * mfc_
            mixed = []
            for b in range(bsz):
                tb = t[b * psz:(b + 1) * psz, :]
                g1 = lax.dot_general(tb, tw1_ref[i],
                                     (((0,), (0,)), ((), ())),
                                     preferred_element_type=F32)
                g1 = jax.nn.gelu(g1 + tb1_ref[i])
                g2 = lax.dot_general(tw2_ref[i], g1,
                                     (((0,), (1,)), ((), ())),
                                     preferred_element_type=F32)
                mixed.append(g2 + tb2_ref[i][:, None])
            mx = mx + jnp.concatenate(mixed, axis=0)
            c = _ln_in(mx, ln2g_ref[i], ln2b_ref[i])
            ch = jax.nn.gelu(
                jnp.dot(c, cw1_ref[i], preferred_element_type=F32)
                + cb1_ref[i])
            mx = mx + (jnp.dot(ch, cw2_ref[i], preferred_element_type=F32)
                       + cb2_ref[i])
        mx = _ln_in(mx, lnfg_ref[...], lnfb_ref[...])
        pooled = []
        for b in range(bsz):
            mb = mfc_[b * psz:(b + 1) * psz, :]
            pr = jnp.sum(mx[b * psz:(b + 1) * psz, :] * mb, axis=0,
                         keepdims=True) / jnp.sum(mb)
            pooled.append(pr)
        pooled = jnp.concatenate(pooled, axis=0)
        hid = jnp.maximum(
            jnp.dot(pooled, dw1_ref[...], preferred_element_type=F32)
            + db1_ref[...], 0.0)
        o_ref[...] = (jnp.dot(hid, dw2_ref[...], preferred_element_type=F32)
                      + db2_ref[...])

    nout = p['dec_W2'].shape[1]
    return pl.pallas_call(
        body,
        out_shape=jax.ShapeDtypeStruct((bsz, nout), F32),
    )(sums, invc, mfc, p['ln1_g'], p['ln1_b'], p['tok_W1'], p['tok_b1'],
      p['tok_W2'], p['tok_b2'], p['ln2_g'], p['ln2_b'], p['ch_W1'],
      p['ch_b1'], p['ch_W2'], p['ch_b2'], p['lnf_g'], p['lnf_b'],
      p['dec_W1'], p['dec_b1'].reshape(1, -1), p['dec_W2'],
      p['dec_b2'].reshape(1, -1))


def kernel(x, edge_attr, edge_index, subgraphs_nodes_mapper,
           combined_subgraphs, subgraphs_edges_mapper, subgraphs_batch,
           mask, params):
    p = params
    nm = subgraphs_nodes_mapper
    em = subgraphs_edges_mapper
    batch = subgraphs_batch
    n_nodes, d = x.shape
    n_sub = nm.shape[0]
    e_cnt = edge_attr.shape[0]
    bsz, psz = mask.shape
    nseg = bsz * psz
    nlg = p['gnn_W1'].shape[0]

    align = _NW * _BLK * 25
    nsubp = ((n_sub + align - 1) // align) * align
    npad = nsubp - n_sub
    np_rows = ((n_nodes + 2047) // 2048 + 1) * 2048

    nm32 = nm.astype(jnp.int32)
    nm_g = jnp.concatenate([nm32, jnp.zeros((npad,), jnp.int32)])
    dummies = n_nodes + (jnp.arange(npad, dtype=jnp.int32)
                         % (np_rows - n_nodes))
    nm_s = jnp.concatenate([nm32, dummies])
    batch2d = jnp.concatenate(
        [batch.astype(jnp.int32),
         jnp.full((npad,), nseg, jnp.int32)]).reshape(nsubp, 1)
    ones_blk = jnp.ones((_BLK, d), F32)

    h0 = _matmul_bias(x, p['W_in'], p['b_in'], 2000)
    e = _matmul_bias(edge_attr, p['W_edge'], p['b_edge'], 4000)
    h = _sc_gather(h0, nm_g)

    ep = ((e_cnt + 2047) // 2048) * 2048
    epad = ep - e_cnt
    erange = jnp.arange(epad, dtype=jnp.int32)
    src_p = jnp.concatenate(
        [combined_subgraphs[0].astype(jnp.int32),
         jnp.zeros((epad,), jnp.int32)])
    dst_p = jnp.concatenate(
        [combined_subgraphs[1].astype(jnp.int32),
         n_sub + (erange % (nsubp - n_sub))])
    em_p = jnp.concatenate([em.astype(jnp.int32),
                            jnp.zeros((epad,), jnp.int32)])

    nchunk = nsubp // _CH
    bucket = dst_p // _CH
    pos = jnp.zeros((ep,), jnp.int32)
    counts = []
    for b in range(nchunk):
        mb = bucket == b
        cs = jnp.cumsum(mb.astype(jnp.int32))
        pos = jnp.where(mb, cs - 1, pos)
        counts.append(cs[-1])
    pc = ((jnp.stack(counts) + 255) // 256) * 256
    base2 = jnp.concatenate([jnp.zeros((1,), jnp.int32), jnp.cumsum(pc)])
    dest = base2[bucket] + pos
    ep2 = ep + nchunk * 256
    arange2 = jnp.arange(ep2, dtype=jnp.int32)
    src_s = jnp.zeros((ep2,), jnp.int32).at[dest].set(
        src_p, unique_indices=True)
    em_s = jnp.zeros((ep2,), jnp.int32).at[dest].set(
        em_p, unique_indices=True)
    dmod = (_CH + (arange2 % 16)).at[dest].set(
        dst_p % _CH, unique_indices=True)
    zpad = jnp.zeros((32 - nchunk,), jnp.int32)
    blo = jnp.concatenate([base2[:nchunk] // 256, zpad])
    bhi = jnp.concatenate([base2[1:nchunk + 1] // 256, zpad])

    cparts = None
    for i in range(nlg):
        if i > 0:
            sums, cst = _segsum_onehot(h, batch2d, nseg)
            invc = (1.0 / jnp.maximum(cst[0], 1.0)).reshape(nseg, 1)
            usub = _usub_relu(sums, invc, p['U_W'][i - 1], p['U_b'][i - 1])
            hp = _expand_add(h, batch2d, usub)
            parts = _sc_scatter_partials(hp, nm_s, ones_blk, np_rows, False)
            if cparts is None:
                cparts = _sc_scatter_partials(hp, nm_s, ones_blk, np_rows,
                                              True)
            table = _combine_mean_table(parts[0], parts[1],
                                        cparts[0], cparts[1])
            h = _sc_gather(table, nm_g)
        agg = _sc_message_agg(h, e, src_s, dmod, em_s, blo, bhi, nsubp)
        y, st = _gnn_mlp_stats(h, agg, p['gnn_W1'][i], p['gnn_b1'][i],
                               p['gnn_W2'][i], p['gnn_b2'][i],
                               p['gnn_eps'][i], n_sub)
        mean = st[0] / n_sub
        var = st[1] / n_sub - mean * mean
        scale = p['bn_g'][i] * lax.rsqrt(var + 1e-5)
        shift = p['bn_b'][i] - mean * scale
        h = _residual_bn_relu(h, y, scale, shift)

    sums, cst = _segsum_onehot(h, batch2d, nseg)
    invc = (1.0 / jnp.maximum(cst[0], 1.0)).reshape(nseg, 1)
    mfc = mask.astype(F32).reshape(nseg, 1)
    return _mixer_head(sums, invc, mfc, bsz, psz, p)
